# Initial kernel scaffold; baseline (speedup 1.0000x reference)
import jax
import jax.numpy as jnp
from jax.experimental import pallas as pl


def kernel(image_tensor, sentence_tensor, conv1_w, conv1_b, conv2_w, conv2_b, conv3_w, conv3_b, emb_table, gru_w_ih, gru_w_hh, gru_b_ih, gru_b_hh, conv1x1_w, conv1x1_b, fc1_w, fc1_b):
    raise NotImplementedError("write your pallas kernel here")


# speedup vs baseline: 1.1995x; 1.1995x over previous
import jax
import jax.numpy as jnp
from jax.experimental import pallas as pl
from jax.experimental.pallas import tpu as pltpu

H = 8
NCLS = 1000
K = 128
L = 50


def _conv2d(x, w, b, pad):
    y = jax.lax.conv_general_dilated(x, w, (1, 1), [(pad, pad), (pad, pad)],
                                     dimension_numbers=('NCHW', 'OIHW', 'NCHW'))
    return y + b[None, :, None, None]


def _avgpool3(x):
    s = jax.lax.reduce_window(x, 0.0, jax.lax.add, (1, 1, 3, 3), (1, 1, 2, 2),
                              [(0, 0), (0, 0), (1, 1), (1, 1)])
    return s / 9.0


def _tail_kernel(rep_ref, fc1_w_ref, fc1_b_ref, logits_ref, mask_ref):
    rep = rep_ref[...]
    presence = jnp.sum(rep * rep, axis=0, keepdims=True)
    m = jnp.sum(rep, axis=0, keepdims=True)

    bits = presence.astype(jnp.float32).view(jnp.int32)

    def body(it, carry):
        lo, hi = carry
        mid = (lo + hi + 1) // 2
        c = jnp.sum((bits >= mid).astype(jnp.int32))
        lo, hi = jax.lax.cond(c >= K, lambda: (mid, hi), lambda: (lo, mid - 1))
        return lo, hi

    lo0 = jnp.int32(0)
    hi0 = jnp.int32(2147483647 - 1)
    lo, hi = jax.lax.fori_loop(0, 31, body, (lo0, hi0))
    thresh = lo
    gt = (bits > thresh).astype(jnp.float32)
    eq = (bits == thresh).astype(jnp.float32)
    n_gt = jnp.sum(gt)
    need = jnp.float32(K) - n_gt
    eq_rank = eq
    for s in (1, 2, 4, 8, 16, 32, 64, 128, 256, 512):
        eq_rank = eq_rank + jnp.concatenate(
            [jnp.zeros((1, s), jnp.float32), eq_rank[:, :-s]], axis=1)
    mask = gt + eq * (eq_rank <= need).astype(jnp.float32)
    mask_ref[...] = mask

    attended = m * mask
    logits = jax.lax.dot_general(attended, fc1_w_ref[...],
                                 (((1,), (1,)), ((), ())),
                                 preferred_element_type=jnp.float32)
    logits = logits + fc1_b_ref[...][None, :]
    mx = jnp.max(logits, axis=1, keepdims=True)
    sh = logits - mx
    lse = jnp.log(jnp.sum(jnp.exp(sh), axis=1, keepdims=True))
    logits_ref[...] = sh - lse


def kernel(image_tensor, sentence_tensor, conv1_w, conv1_b, conv2_w, conv2_b,
           conv3_w, conv3_b, emb_table, gru_w_ih, gru_w_hh, gru_b_ih, gru_b_hh,
           conv1x1_w, conv1x1_b, fc1_w, fc1_b):
    x = jax.nn.relu(_avgpool3(_conv2d(image_tensor, conv1_w, conv1_b, 2)))
    x = jax.nn.relu(_avgpool3(_conv2d(x, conv2_w, conv2_b, 2)))
    x = jax.nn.relu(_avgpool3(_conv2d(x, conv3_w, conv3_b, 2)))

    emb = jnp.take(emb_table, sentence_tensor, axis=0)

    def step(h, e):
        gi = e @ gru_w_ih.T + gru_b_ih
        gh = h @ gru_w_hh.T + gru_b_hh
        i_r, i_z, i_n = jnp.split(gi, 3)
        h_r, h_z, h_n = jnp.split(gh, 3)
        r = jax.nn.sigmoid(i_r + h_r)
        z = jax.nn.sigmoid(i_z + h_z)
        n = jnp.tanh(i_n + r * h_n)
        return (1.0 - z) * n + z * h, None

    h, _ = jax.lax.scan(step, jnp.zeros((H,), jnp.float32), emb)
    enc_sent = h.reshape(H, 1, 1)

    encoded_sum = x + enc_sent
    rep = _conv2d(encoded_sum, conv1x1_w, conv1x1_b, 0)[0]
    rep2 = rep.reshape(2, 1024)

    class_preds, latent_mask = pl.pallas_call(
        _tail_kernel,
        out_shape=(jax.ShapeDtypeStruct((1, NCLS), jnp.float32),
                   jax.ShapeDtypeStruct((1, 1024), jnp.float32)),
    )(rep2, fc1_w, fc1_b)
    return class_preds, latent_mask.reshape(1024)
